# baseline (speedup 1.0000x reference)
import jax
import jax.numpy as jnp
from jax.experimental import pallas as pl
from jax.experimental.pallas import tpu as pltpu

MAP_H, MAP_W = 128, 64
NC = 8


def _som_kernel(x_ref, w_hbm, out_ref, wv_ref, e_ref, sems):
    bkc = wv_ref.shape[1]
    copies = [
        pltpu.make_async_copy(
            w_hbm.at[pl.ds(i * bkc, bkc), :], wv_ref.at[i], sems.at[i]
        )
        for i in range(NC)
    ]
    for c in copies:
        c.start()
    x = x_ref[...]
    xn2 = jnp.sum(x * x, axis=1, keepdims=True)
    ones_d = jnp.ones((1, x.shape[1]), dtype=jnp.float32)
    ms = []
    es = []
    for i in range(NC):
        copies[i].wait()
        w = wv_ref[i]
        xw = jax.lax.dot_general(
            x, w, (((1,), (1,)), ((), ())), preferred_element_type=jnp.float32
        )
        wn2 = jax.lax.dot_general(
            ones_d, w * w, (((1,), (1,)), ((), ())),
            preferred_element_type=jnp.float32,
        )
        d2 = jnp.maximum(wn2 + xn2 - 2.0 * xw, 0.0)
        s = -10.0 * jnp.sqrt(d2)
        m_i = jnp.max(s, axis=1, keepdims=True)
        e = jnp.exp(s - m_i)
        e_ref[:, i * bkc:(i + 1) * bkc] = e
        ms.append(m_i)
        es.append(jnp.sum(e, axis=1, keepdims=True))
    m = ms[0]
    for m_i in ms[1:]:
        m = jnp.maximum(m, m_i)
    denom = jnp.zeros_like(m)
    scales = []
    for m_i, e_i in zip(ms, es):
        sc = jnp.exp(m_i - m)
        denom = denom + sc * e_i
        scales.append(sc)
    inv = 1.0 / denom
    for i in range(NC):
        out_ref[:, i * bkc:(i + 1) * bkc] = (
            e_ref[:, i * bkc:(i + 1) * bkc] * (scales[i] * inv)
        )


def kernel(input_signal, weight_matrix):
    b, d = input_signal.shape
    kk = weight_matrix.shape[0]
    bkc = kk // NC
    out = pl.pallas_call(
        _som_kernel,
        in_specs=[
            pl.BlockSpec((b, d), lambda: (0, 0)),
            pl.BlockSpec(memory_space=pltpu.MemorySpace.HBM),
        ],
        out_specs=pl.BlockSpec((b, kk), lambda: (0, 0)),
        out_shape=jax.ShapeDtypeStruct((b, kk), jnp.float32),
        scratch_shapes=[
            pltpu.VMEM((NC, bkc, d), jnp.float32),
            pltpu.VMEM((b, kk), jnp.float32),
            pltpu.SemaphoreType.DMA((NC,)),
        ],
    )(input_signal, weight_matrix)
    return out.reshape(b, MAP_H, MAP_W)


# speedup vs baseline: 1.0114x; 1.0114x over previous
import jax
import jax.numpy as jnp
from jax.experimental import pallas as pl
from jax.experimental.pallas import tpu as pltpu

MAP_H, MAP_W = 128, 64
NC = 16


def _som_kernel(x_ref, w_hbm, out_ref, wv_ref, s_ref, sems):
    bkc = wv_ref.shape[1]
    copies = [
        pltpu.make_async_copy(
            w_hbm.at[pl.ds(i * bkc, bkc), :], wv_ref.at[i], sems.at[i]
        )
        for i in range(NC)
    ]
    for c in copies:
        c.start()
    x = x_ref[...]
    xn2 = jnp.sum(x * x, axis=1, keepdims=True)
    ones_d = jnp.ones((1, x.shape[1]), dtype=jnp.float32)
    for i in range(NC):
        copies[i].wait()
        w = wv_ref[i]
        xw = jax.lax.dot_general(
            x, w, (((1,), (1,)), ((), ())), preferred_element_type=jnp.float32
        )
        wn2 = jax.lax.dot_general(
            ones_d, w * w, (((1,), (1,)), ((), ())),
            preferred_element_type=jnp.float32,
        )
        d2 = jnp.maximum(wn2 + xn2 - 2.0 * xw, 0.0)
        s_ref[:, i * bkc:(i + 1) * bkc] = -10.0 * jnp.sqrt(d2)
    s = s_ref[...]
    m = jnp.max(s, axis=1, keepdims=True)
    e = jnp.exp(s - m)
    out_ref[...] = e / jnp.sum(e, axis=1, keepdims=True)


def kernel(input_signal, weight_matrix):
    b, d = input_signal.shape
    kk = weight_matrix.shape[0]
    bkc = kk // NC
    out = pl.pallas_call(
        _som_kernel,
        in_specs=[
            pl.BlockSpec((b, d), lambda: (0, 0)),
            pl.BlockSpec(memory_space=pltpu.MemorySpace.HBM),
        ],
        out_specs=pl.BlockSpec((b, kk), lambda: (0, 0)),
        out_shape=jax.ShapeDtypeStruct((b, kk), jnp.float32),
        scratch_shapes=[
            pltpu.VMEM((NC, bkc, d), jnp.float32),
            pltpu.VMEM((b, kk), jnp.float32),
            pltpu.SemaphoreType.DMA((NC,)),
        ],
    )(input_signal, weight_matrix)
    return out.reshape(b, MAP_H, MAP_W)


# speedup vs baseline: 1.1360x; 1.1232x over previous
import jax
import jax.numpy as jnp
from jax.experimental import pallas as pl
from jax.experimental.pallas import tpu as pltpu

MAP_H, MAP_W = 128, 64
NC = 4


def _som_kernel(x_ref, w_hbm, out_ref, wv_ref, s_ref, sems):
    bkc = wv_ref.shape[1]
    copies = [
        pltpu.make_async_copy(
            w_hbm.at[pl.ds(i * bkc, bkc), :], wv_ref.at[i], sems.at[i]
        )
        for i in range(NC)
    ]
    for c in copies:
        c.start()
    x = x_ref[...]
    xn2 = jnp.sum(x * x, axis=1, keepdims=True)
    ones_d = jnp.ones((1, x.shape[1]), dtype=jnp.float32)
    for i in range(NC):
        copies[i].wait()
        w = wv_ref[i]
        xw = jax.lax.dot_general(
            x, w, (((1,), (1,)), ((), ())), preferred_element_type=jnp.float32
        )
        wn2 = jax.lax.dot_general(
            ones_d, w * w, (((1,), (1,)), ((), ())),
            preferred_element_type=jnp.float32,
        )
        d2 = jnp.maximum(wn2 + xn2 - 2.0 * xw, 0.0)
        s_ref[:, i * bkc:(i + 1) * bkc] = -10.0 * jnp.sqrt(d2)
    s = s_ref[...]
    m = jnp.max(s, axis=1, keepdims=True)
    e = jnp.exp(s - m)
    out_ref[...] = e / jnp.sum(e, axis=1, keepdims=True)


def kernel(input_signal, weight_matrix):
    b, d = input_signal.shape
    kk = weight_matrix.shape[0]
    bkc = kk // NC
    out = pl.pallas_call(
        _som_kernel,
        in_specs=[
            pl.BlockSpec((b, d), lambda: (0, 0)),
            pl.BlockSpec(memory_space=pltpu.MemorySpace.HBM),
        ],
        out_specs=pl.BlockSpec((b, kk), lambda: (0, 0)),
        out_shape=jax.ShapeDtypeStruct((b, kk), jnp.float32),
        scratch_shapes=[
            pltpu.VMEM((NC, bkc, d), jnp.float32),
            pltpu.VMEM((b, kk), jnp.float32),
            pltpu.SemaphoreType.DMA((NC,)),
        ],
    )(input_signal, weight_matrix)
    return out.reshape(b, MAP_H, MAP_W)
